# baseline (speedup 1.0000x reference)
import functools

import jax
import jax.numpy as jnp
from jax import lax
from jax.experimental import pallas as pl
from jax.experimental.pallas import tpu as pltpu
from jax.experimental.pallas import tpu_sc as plsc

_CHUNK_B = 2
_PAD = 104


def _pool_kernel_body(CPW, S, E, table_hbm, idx_hbm, out_hbm,
                      idx_v, rows_v, pool_v, sem):
    NC = 2
    BPW = CPW * _CHUNK_B
    wid = lax.axis_index("s") * NC + lax.axis_index("c")

    pltpu.sync_copy(idx_hbm.at[pl.ds(wid * CPW * _PAD, CPW * _PAD)], idx_v)

    def _gather(c, par):
        pltpu.async_copy(
            table_hbm.at[idx_v.at[pl.ds(c * _PAD, _PAD)]],
            rows_v.at[pl.ds(par * _PAD, _PAD)],
            sem,
        )

    _gather(0, 0)
    _gather(1, 1)

    inv_s = 1.0 / S
    nhalf = E // 32

    def chunk_body(c, carry):
        par = lax.rem(c, 2)
        base = par * _PAD
        pltpu.make_async_copy(
            table_hbm.at[idx_v.at[pl.ds(0, _PAD)]],
            rows_v.at[pl.ds(0, _PAD)],
            sem,
        ).wait()

        @pl.when(c + 2 < CPW)
        def _():
            _gather(c + 2, par)

        for bag in range(_CHUNK_B):
            accs = [
                [jnp.zeros((16,), jnp.float32) for _ in range(2)]
                for _ in range(nhalf)
            ]
            for s in range(S):
                r = base + bag * S + s
                for h in range(nhalf):
                    a, b2 = plsc.unpack(
                        rows_v[r, pl.ds(h * 32, 32)],
                        format=plsc.PackFormat.INTERLEAVED,
                    )
                    accs[h][0] = accs[h][0] + a
                    accs[h][1] = accs[h][1] + b2
            row_out = c * _CHUNK_B + bag
            for h in range(nhalf):
                packed = plsc.pack(
                    accs[h][0] * inv_s,
                    accs[h][1] * inv_s,
                    format=plsc.PackFormat.INTERLEAVED,
                )
                pool_v[row_out, pl.ds(h * 32, 32)] = packed
        return carry

    lax.fori_loop(0, CPW, chunk_body, 0, unroll=1)

    pltpu.sync_copy(pool_v, out_hbm.at[pl.ds(wid * BPW, BPW)])


def _make_pool_kernel(B, S, E, NW):
    BPW = B // NW
    CPW = BPW // _CHUNK_B
    mesh = plsc.VectorSubcoreMesh(core_axis_name="c", subcore_axis_name="s")
    return pl.kernel(
        functools.partial(_pool_kernel_body, CPW, S, E),
        out_type=jax.ShapeDtypeStruct((B, E), jnp.bfloat16),
        mesh=mesh,
        scratch_types=[
            pltpu.VMEM((CPW * _PAD,), jnp.int32),
            pltpu.VMEM((2 * _PAD, E), jnp.bfloat16),
            pltpu.VMEM((BPW, E), jnp.bfloat16),
            pltpu.SemaphoreType.DMA,
        ],
        compiler_params=pltpu.CompilerParams(
            use_tc_tiling_on_sc=False, needs_layout_passes=False
        ),
    )


def _mm_body(x_ref, w_ref, b_ref, o_ref):
    o_ref[...] = (
        jnp.dot(
            x_ref[...].astype(jnp.float32),
            w_ref[...],
            preferred_element_type=jnp.float32,
        )
        + b_ref[...]
    )


def kernel(inputs, table, W, b):
    B, S = inputs.shape
    V, E = table.shape
    O = W.shape[0]
    NW = 32

    tbl_bf = table.astype(jnp.bfloat16)
    idx2 = inputs.astype(jnp.int32).reshape(B // _CHUNK_B, _CHUNK_B * S)
    idx_flat = jnp.pad(idx2, ((0, 0), (0, _PAD - _CHUNK_B * S))).reshape(-1)

    pooled = _make_pool_kernel(B, S, E, NW)(tbl_bf, idx_flat)

    out = pl.pallas_call(
        _mm_body,
        out_shape=jax.ShapeDtypeStruct((B, O), jnp.float32),
    )(pooled, W.T, b.reshape(1, O))
    return out


# speedup vs baseline: 1.0720x; 1.0720x over previous
import functools

import jax
import jax.numpy as jnp
from jax import lax
from jax.experimental import pallas as pl
from jax.experimental.pallas import tpu as pltpu
from jax.experimental.pallas import tpu_sc as plsc

_VBLK = 512


def _tp_body(xt_ref, o_ref):
    x = xt_ref[...]
    y = jnp.transpose(x, (1, 0))
    o_ref[:, : x.shape[0]] = y
    o_ref[:, x.shape[0]:] = jnp.zeros_like(y)


def _stage_table(table_t, V, E):
    nblk = (V + _VBLK - 1) // _VBLK
    return pl.pallas_call(
        _tp_body,
        grid=(nblk,),
        in_specs=[pl.BlockSpec((E, _VBLK), lambda i: (0, i))],
        out_specs=pl.BlockSpec((_VBLK, 2 * E), lambda i: (i, 0)),
        out_shape=jax.ShapeDtypeStruct((V, 2 * E), jnp.float32),
    )(table_t)


def _pool_kernel_body(CPW, CHUNK_B, S, E, table_hbm, idx_hbm, out_hbm,
                      idx_v, rows_v, pool_v, sem):
    NC = 2
    CHUNK_IDX = CHUNK_B * S
    RPAD = 104
    BPW = CPW * CHUNK_B
    wid = lax.axis_index("s") * NC + lax.axis_index("c")

    pltpu.sync_copy(idx_hbm.at[wid], idx_v)

    def _gather(c, par):
        pltpu.async_copy(
            table_hbm.at[idx_v.at[c]],
            rows_v.at[pl.ds(par * RPAD, CHUNK_IDX)],
            sem,
        )

    _gather(0, 0)
    _gather(1, 1)

    nvec = E // 16

    def chunk_body(c, carry):
        par = lax.rem(c, 2)
        base = par * RPAD
        pltpu.make_async_copy(
            table_hbm.at[idx_v.at[c]], rows_v.at[pl.ds(0, CHUNK_IDX)], sem
        ).wait()

        @pl.when(c + 2 < CPW)
        def _():
            _gather(c + 2, par)

        for bag in range(CHUNK_B):
            accs = [jnp.zeros((16,), jnp.float32) for _ in range(nvec)]
            for s in range(S):
                r = base + bag * S + s
                for j in range(nvec):
                    accs[j] = accs[j] + rows_v[r, pl.ds(j * 16, 16)]
            row_out = c * CHUNK_B + bag
            for j in range(nvec):
                pool_v[row_out, pl.ds(j * 16, 16)] = accs[j] * (1.0 / S)
        return carry

    lax.fori_loop(0, CPW, chunk_body, 0, unroll=1)

    pltpu.sync_copy(pool_v, out_hbm.at[pl.ds(wid * BPW, BPW)])


def _make_pool_kernel(B, S, E, V, NW, CHUNK_B):
    BPW = B // NW
    CPW = BPW // CHUNK_B
    CHUNK_IDX = CHUNK_B * S
    mesh = plsc.VectorSubcoreMesh(core_axis_name="c", subcore_axis_name="s")
    return pl.kernel(
        functools.partial(_pool_kernel_body, CPW, CHUNK_B, S, E),
        out_type=jax.ShapeDtypeStruct((B, E), jnp.float32),
        mesh=mesh,
        scratch_types=[
            pltpu.VMEM((CPW, CHUNK_IDX), jnp.int32),
            pltpu.VMEM((2 * 104, 2 * E), jnp.float32),
            pltpu.VMEM((BPW, E), jnp.float32),
            pltpu.SemaphoreType.DMA,
        ],
        compiler_params=pltpu.CompilerParams(use_tc_tiling_on_sc=True),
    )


def _mm_body(x_ref, w_ref, b_ref, o_ref):
    o_ref[...] = (
        jnp.dot(x_ref[...], w_ref[...], preferred_element_type=jnp.float32)
        + b_ref[...]
    )


def kernel(inputs, table, W, b):
    B, S = inputs.shape
    V, E = table.shape
    O = W.shape[0]
    NW = 32
    CHUNK_B = 2
    BPW = B // NW
    CPW = BPW // CHUNK_B

    tbl_stage = _stage_table(table.T, V, E)
    idx3 = inputs.astype(jnp.int32).reshape(NW, CPW, CHUNK_B * S)
    pooled = _make_pool_kernel(B, S, E, V, NW, CHUNK_B)(tbl_stage, idx3)

    out = pl.pallas_call(
        _mm_body,
        out_shape=jax.ShapeDtypeStruct((B, O), jnp.float32),
    )(pooled, W.T, b.reshape(1, O))
    return out


# speedup vs baseline: 1.7191x; 1.6037x over previous
import functools

import jax
import jax.numpy as jnp
from jax import lax
from jax.experimental import pallas as pl
from jax.experimental.pallas import tpu as pltpu
from jax.experimental.pallas import tpu_sc as plsc

_VBLK = 2048


def _tp_body(xt_ref, o_ref):
    x = xt_ref[...]
    E = x.shape[0]
    eye = jnp.eye(E, dtype=jnp.float32)
    y = jax.lax.dot_general(
        x, eye, (((0,), (0,)), ((), ())),
        preferred_element_type=jnp.float32,
    )
    o_ref[:, :E] = y
    o_ref[:, E:] = jnp.zeros_like(y)


def _stage_table(table_t, V, E):
    nblk = (V + _VBLK - 1) // _VBLK
    return pl.pallas_call(
        _tp_body,
        grid=(nblk,),
        in_specs=[pl.BlockSpec((E, _VBLK), lambda i: (0, i))],
        out_specs=pl.BlockSpec((_VBLK, 2 * E), lambda i: (i, 0)),
        out_shape=jax.ShapeDtypeStruct((V, 2 * E), jnp.float32),
    )(table_t)


def _pool_kernel_body(CPW, CHUNK_B, S, E, table_hbm, idx_hbm, out_hbm,
                      idx_v, rows_v, pool_v, sem):
    NC = 2
    CHUNK_IDX = CHUNK_B * S
    RPAD = 104
    BPW = CPW * CHUNK_B
    wid = lax.axis_index("s") * NC + lax.axis_index("c")

    pltpu.sync_copy(idx_hbm.at[wid], idx_v)

    def _gather(c, par):
        pltpu.async_copy(
            table_hbm.at[idx_v.at[c]],
            rows_v.at[pl.ds(par * RPAD, CHUNK_IDX)],
            sem,
        )

    _gather(0, 0)
    _gather(1, 1)

    nvec = E // 16

    def chunk_body(c, carry):
        par = lax.rem(c, 2)
        base = par * RPAD
        pltpu.make_async_copy(
            table_hbm.at[idx_v.at[c]], rows_v.at[pl.ds(0, CHUNK_IDX)], sem
        ).wait()

        @pl.when(c + 2 < CPW)
        def _():
            _gather(c + 2, par)

        for bag in range(CHUNK_B):
            accs = [jnp.zeros((16,), jnp.float32) for _ in range(nvec)]
            for s in range(S):
                r = base + bag * S + s
                for j in range(nvec):
                    accs[j] = accs[j] + rows_v[r, pl.ds(j * 16, 16)]
            row_out = c * CHUNK_B + bag
            for j in range(nvec):
                pool_v[row_out, pl.ds(j * 16, 16)] = accs[j] * (1.0 / S)
        return carry

    lax.fori_loop(0, CPW, chunk_body, 0, unroll=1)

    pltpu.sync_copy(pool_v, out_hbm.at[pl.ds(wid * BPW, BPW)])


def _make_pool_kernel(B, S, E, V, NW, CHUNK_B):
    BPW = B // NW
    CPW = BPW // CHUNK_B
    CHUNK_IDX = CHUNK_B * S
    mesh = plsc.VectorSubcoreMesh(core_axis_name="c", subcore_axis_name="s")
    return pl.kernel(
        functools.partial(_pool_kernel_body, CPW, CHUNK_B, S, E),
        out_type=jax.ShapeDtypeStruct((B, E), jnp.float32),
        mesh=mesh,
        scratch_types=[
            pltpu.VMEM((CPW, CHUNK_IDX), jnp.int32),
            pltpu.VMEM((2 * 104, 2 * E), jnp.float32),
            pltpu.VMEM((BPW, E), jnp.float32),
            pltpu.SemaphoreType.DMA,
        ],
        compiler_params=pltpu.CompilerParams(use_tc_tiling_on_sc=True),
    )


def _mm_body(x_ref, w_ref, b_ref, o_ref):
    o_ref[...] = (
        jnp.dot(x_ref[...], w_ref[...], preferred_element_type=jnp.float32)
        + b_ref[...]
    )


def kernel(inputs, table, W, b):
    B, S = inputs.shape
    V, E = table.shape
    O = W.shape[0]
    NW = 32
    CHUNK_B = 2
    BPW = B // NW
    CPW = BPW // CHUNK_B

    tbl_stage = _stage_table(table.T, V, E)
    idx3 = inputs.astype(jnp.int32).reshape(NW, CPW, CHUNK_B * S)
    pooled = _make_pool_kernel(B, S, E, V, NW, CHUNK_B)(tbl_stage, idx3)

    out = pl.pallas_call(
        _mm_body,
        out_shape=jax.ShapeDtypeStruct((B, O), jnp.float32),
    )(pooled, W.T, b.reshape(1, O))
    return out


# speedup vs baseline: 1.9333x; 1.1246x over previous
import functools

import jax
import jax.numpy as jnp
from jax import lax
from jax.experimental import pallas as pl
from jax.experimental.pallas import tpu as pltpu
from jax.experimental.pallas import tpu_sc as plsc

_HALF = 1024
_VBLK = 2 * _HALF


def _tp_body(xt_ref, o_ref):
    x = xt_ref[...]
    E = x.shape[0]
    eye = jnp.eye(E, dtype=jnp.float32)

    def dot_t(m):
        return jax.lax.dot_general(
            m, eye, (((0,), (0,)), ((), ())),
            preferred_element_type=jnp.float32,
        )

    o_ref[:, :E] = dot_t(x[:, :_HALF])
    o_ref[:, E:] = dot_t(x[:, _HALF:])


def _stage_table(table_t, V, E):
    nblk = (V + _VBLK - 1) // _VBLK
    return pl.pallas_call(
        _tp_body,
        grid=(nblk,),
        in_specs=[pl.BlockSpec((E, _VBLK), lambda i: (0, i))],
        out_specs=pl.BlockSpec((_HALF, 2 * E), lambda i: (i, 0)),
        out_shape=jax.ShapeDtypeStruct((nblk * _HALF, 2 * E), jnp.float32),
    )(table_t)


def _pool_kernel_body(CPW, CHUNK_B, S, E, table_hbm, idx_hbm, out_hbm,
                      idx_v, rows_v, pool_v, sem):
    NC = 2
    CHUNK_IDX = CHUNK_B * S
    BPW = CPW * CHUNK_B
    wid = lax.axis_index("s") * NC + lax.axis_index("c")

    pltpu.sync_copy(idx_hbm.at[wid], idx_v)

    def _gather(c, par):
        pltpu.async_copy(
            table_hbm.at[idx_v.at[c]],
            rows_v.at[pl.ds(par * CHUNK_IDX, CHUNK_IDX)],
            sem,
        )

    _gather(0, 0)
    _gather(1, 1)

    nvec = E // 16

    def chunk_body(c, carry):
        par = lax.rem(c, 2)
        base = par * CHUNK_IDX
        pltpu.make_async_copy(
            table_hbm.at[idx_v.at[c]], rows_v.at[pl.ds(0, CHUNK_IDX)], sem
        ).wait()

        @pl.when(c + 2 < CPW)
        def _():
            _gather(c + 2, par)

        for bag in range(CHUNK_B):
            accs = [jnp.zeros((16,), jnp.float32) for _ in range(nvec)]
            for s in range(S):
                r = base + bag * S + s
                for j in range(nvec):
                    accs[j] = accs[j] + rows_v[r, pl.ds(j * 16, 16)]
            row_out = c * CHUNK_B + bag
            for j in range(nvec):
                pool_v[row_out, pl.ds(j * 16, 16)] = accs[j] * (1.0 / S)
        return carry

    lax.fori_loop(0, CPW, chunk_body, 0, unroll=1)

    pltpu.sync_copy(pool_v, out_hbm.at[pl.ds(wid * BPW, BPW)])


def _make_pool_kernel(B, S, E, Vp, NW, CHUNK_B):
    BPW = B // NW
    CPW = BPW // CHUNK_B
    CHUNK_IDX = CHUNK_B * S
    mesh = plsc.VectorSubcoreMesh(core_axis_name="c", subcore_axis_name="s")
    return pl.kernel(
        functools.partial(_pool_kernel_body, CPW, CHUNK_B, S, E),
        out_type=jax.ShapeDtypeStruct((B, E), jnp.float32),
        mesh=mesh,
        scratch_types=[
            pltpu.VMEM((CPW, CHUNK_IDX), jnp.int32),
            pltpu.VMEM((2 * CHUNK_IDX, E), jnp.float32),
            pltpu.VMEM((BPW, E), jnp.float32),
            pltpu.SemaphoreType.DMA,
        ],
        compiler_params=pltpu.CompilerParams(use_tc_tiling_on_sc=False),
    )


def _mm_body(x_ref, w_ref, b_ref, o_ref):
    o_ref[...] = (
        jnp.dot(x_ref[...], w_ref[...], preferred_element_type=jnp.float32)
        + b_ref[...]
    )


def kernel(inputs, table, W, b):
    B, S = inputs.shape
    V, E = table.shape
    O = W.shape[0]
    NW = 32
    CHUNK_B = 2
    BPW = B // NW
    CPW = BPW // CHUNK_B

    stage = _stage_table(table.T, V, E)
    Vp = stage.shape[0] * 2
    tbl_lin = stage.reshape(Vp, E)

    v = inputs.astype(jnp.int32)
    q = v & (_VBLK - 1)
    vp = (v & ~(_VBLK - 1)) | ((q & (_HALF - 1)) << 1) | (q >> 10)
    idx3 = vp.reshape(NW, CPW, CHUNK_B * S)

    pooled = _make_pool_kernel(B, S, E, Vp, NW, CHUNK_B)(tbl_lin, idx3)

    out = pl.pallas_call(
        _mm_body,
        out_shape=jax.ShapeDtypeStruct((B, O), jnp.float32),
    )(pooled, W.T, b.reshape(1, O))
    return out


# speedup vs baseline: 2.3275x; 1.2039x over previous
import functools

import jax
import jax.numpy as jnp
from jax import lax
from jax.experimental import pallas as pl
from jax.experimental.pallas import tpu as pltpu
from jax.experimental.pallas import tpu_sc as plsc

_HALF = 4096
_HBITS = _HALF.bit_length() - 1
_VBLK = 2 * _HALF


def _tp_body(xt_ref, o_ref):
    x = xt_ref[...]
    E = x.shape[0]
    eye = jnp.eye(E, dtype=jnp.float32)

    def dot_t(m):
        return jax.lax.dot_general(
            m, eye, (((0,), (0,)), ((), ())),
            preferred_element_type=jnp.float32,
        )

    o_ref[...] = jnp.concatenate(
        [dot_t(x[:, :_HALF]), dot_t(x[:, _HALF:])], axis=1
    )


def _stage_table(table_t, V, E):
    nblk = (V + _VBLK - 1) // _VBLK
    return pl.pallas_call(
        _tp_body,
        grid=(nblk,),
        in_specs=[pl.BlockSpec((E, _VBLK), lambda i: (0, i))],
        out_specs=pl.BlockSpec((_HALF, 2 * E), lambda i: (i, 0)),
        out_shape=jax.ShapeDtypeStruct((nblk * _HALF, 2 * E), jnp.float32),
    )(table_t)


def _pool_kernel_body(CPW, CHUNK_B, S, E, table_hbm, idx_hbm, out_hbm,
                      idx_v, rows_v, pool_v, sem):
    NC = 2
    CHUNK_IDX = CHUNK_B * S
    BPW = CPW * CHUNK_B
    wid = lax.axis_index("s") * NC + lax.axis_index("c")

    pltpu.sync_copy(idx_hbm.at[wid], idx_v)

    def _gather(c, par):
        pltpu.async_copy(
            table_hbm.at[idx_v.at[c]],
            rows_v.at[pl.ds(par * CHUNK_IDX, CHUNK_IDX)],
            sem,
        )

    _gather(0, 0)
    _gather(1, 1)

    nvec = E // 16

    def chunk_body(c, carry):
        par = lax.rem(c, 2)
        base = par * CHUNK_IDX
        pltpu.make_async_copy(
            table_hbm.at[idx_v.at[c]], rows_v.at[pl.ds(0, CHUNK_IDX)], sem
        ).wait()

        @pl.when(c + 2 < CPW)
        def _():
            _gather(c + 2, par)

        for bag in range(CHUNK_B):
            accs = [jnp.zeros((16,), jnp.float32) for _ in range(nvec)]
            for s in range(S):
                r = base + bag * S + s
                for j in range(nvec):
                    accs[j] = accs[j] + rows_v[r, pl.ds(j * 16, 16)]
            row_out = c * CHUNK_B + bag
            for j in range(nvec):
                pool_v[row_out, pl.ds(j * 16, 16)] = accs[j] * (1.0 / S)
        return carry

    lax.fori_loop(0, CPW, chunk_body, 0, unroll=1)

    pltpu.sync_copy(pool_v, out_hbm.at[pl.ds(wid * BPW, BPW)])


def _make_pool_kernel(B, S, E, Vp, NW, CHUNK_B):
    BPW = B // NW
    CPW = BPW // CHUNK_B
    CHUNK_IDX = CHUNK_B * S
    mesh = plsc.VectorSubcoreMesh(core_axis_name="c", subcore_axis_name="s")
    return pl.kernel(
        functools.partial(_pool_kernel_body, CPW, CHUNK_B, S, E),
        out_type=jax.ShapeDtypeStruct((B, E), jnp.float32),
        mesh=mesh,
        scratch_types=[
            pltpu.VMEM((CPW, CHUNK_IDX), jnp.int32),
            pltpu.VMEM((2 * CHUNK_IDX, E), jnp.float32),
            pltpu.VMEM((BPW, E), jnp.float32),
            pltpu.SemaphoreType.DMA,
        ],
        compiler_params=pltpu.CompilerParams(use_tc_tiling_on_sc=False),
    )


def _mm_body(x_ref, w_ref, b_ref, o_ref):
    o_ref[...] = (
        jnp.dot(x_ref[...], w_ref[...], preferred_element_type=jnp.float32)
        + b_ref[...]
    )


def kernel(inputs, table, W, b):
    B, S = inputs.shape
    V, E = table.shape
    O = W.shape[0]
    NW = 32
    CHUNK_B = 2
    BPW = B // NW
    CPW = BPW // CHUNK_B

    stage = _stage_table(table.T, V, E)
    Vp = stage.shape[0] * 2
    tbl_lin = stage.reshape(Vp, E)

    v = inputs.astype(jnp.int32)
    q = v & (_VBLK - 1)
    vp = (v & ~(_VBLK - 1)) | ((q & (_HALF - 1)) << 1) | (q >> _HBITS)
    idx3 = vp.reshape(NW, CPW, CHUNK_B * S)

    pooled = _make_pool_kernel(B, S, E, Vp, NW, CHUNK_B)(tbl_lin, idx3)

    out = pl.pallas_call(
        _mm_body,
        out_shape=jax.ShapeDtypeStruct((B, O), jnp.float32),
    )(pooled, W.T, b.reshape(1, O))
    return out


# speedup vs baseline: 2.4501x; 1.0527x over previous
import functools

import jax
import jax.numpy as jnp
from jax import lax
from jax.experimental import pallas as pl
from jax.experimental.pallas import tpu as pltpu
from jax.experimental.pallas import tpu_sc as plsc

_HALF = 4096
_HBITS = _HALF.bit_length() - 1
_VBLK = 2 * _HALF


def _tp_body(xt_ref, o_ref):
    x = xt_ref[...]
    E = x.shape[0]
    eye = jnp.eye(E, dtype=jnp.float32)

    def dot_t(m):
        return jax.lax.dot_general(
            m, eye, (((0,), (0,)), ((), ())),
            preferred_element_type=jnp.float32,
        )

    o_ref[...] = jnp.concatenate(
        [dot_t(x[:, :_HALF]), dot_t(x[:, _HALF:])], axis=1
    )


def _stage_table(table_t, V, E):
    nblk = (V + _VBLK - 1) // _VBLK
    return pl.pallas_call(
        _tp_body,
        grid=(nblk,),
        in_specs=[pl.BlockSpec((E, _VBLK), lambda i: (0, i))],
        out_specs=pl.BlockSpec((_HALF, 2 * E), lambda i: (i, 0)),
        out_shape=jax.ShapeDtypeStruct((nblk * _HALF, 2 * E), jnp.float32),
    )(table_t)


def _pool_kernel_body(CPW, CHUNK_B, S, E, table_hbm, idx_hbm, out_hbm,
                      idx_v, rows_v, pool_v, sem):
    NC = 2
    CHUNK_IDX = CHUNK_B * S
    BPW = CPW * CHUNK_B
    wid = lax.axis_index("s") * NC + lax.axis_index("c")

    pltpu.sync_copy(idx_hbm.at[wid], idx_v)

    def _gather(c, par):
        pltpu.async_copy(
            table_hbm.at[idx_v.at[c]],
            rows_v.at[pl.ds(par * CHUNK_IDX, CHUNK_IDX)],
            sem,
        )

    NBUF = 4
    for p in range(NBUF - 1):
        _gather(p, p)

    nvec = E // 16

    def chunk_body(c, carry):
        par = lax.rem(c, NBUF)
        base = par * CHUNK_IDX
        pltpu.make_async_copy(
            table_hbm.at[idx_v.at[c]], rows_v.at[pl.ds(0, CHUNK_IDX)], sem
        ).wait()

        @pl.when(c + NBUF - 1 < CPW)
        def _():
            _gather(c + NBUF - 1, lax.rem(c + NBUF - 1, NBUF))

        for bag in range(CHUNK_B):
            accs = [jnp.zeros((16,), jnp.float32) for _ in range(nvec)]
            for s in range(S):
                r = base + bag * S + s
                for j in range(nvec):
                    accs[j] = accs[j] + rows_v[r, pl.ds(j * 16, 16)]
            row_out = c * CHUNK_B + bag
            for j in range(nvec):
                pool_v[row_out, pl.ds(j * 16, 16)] = accs[j] * (1.0 / S)
        return carry

    lax.fori_loop(0, CPW, chunk_body, 0, unroll=1)

    pltpu.sync_copy(pool_v, out_hbm.at[pl.ds(wid * BPW, BPW)])


def _make_pool_kernel(B, S, E, Vp, NW, CHUNK_B):
    BPW = B // NW
    CPW = BPW // CHUNK_B
    CHUNK_IDX = CHUNK_B * S
    mesh = plsc.VectorSubcoreMesh(core_axis_name="c", subcore_axis_name="s")
    return pl.kernel(
        functools.partial(_pool_kernel_body, CPW, CHUNK_B, S, E),
        out_type=jax.ShapeDtypeStruct((B, E), jnp.float32),
        mesh=mesh,
        scratch_types=[
            pltpu.VMEM((CPW, CHUNK_IDX), jnp.int32),
            pltpu.VMEM((4 * CHUNK_IDX, E), jnp.float32),
            pltpu.VMEM((BPW, E), jnp.float32),
            pltpu.SemaphoreType.DMA,
        ],
        compiler_params=pltpu.CompilerParams(use_tc_tiling_on_sc=False),
    )


def _mm_body(x_ref, w_ref, b_ref, o_ref):
    o_ref[...] = (
        jnp.dot(x_ref[...], w_ref[...], preferred_element_type=jnp.float32)
        + b_ref[...]
    )


def kernel(inputs, table, W, b):
    B, S = inputs.shape
    V, E = table.shape
    O = W.shape[0]
    NW = 32
    CHUNK_B = 2
    BPW = B // NW
    CPW = BPW // CHUNK_B

    stage = _stage_table(table.T, V, E)
    Vp = stage.shape[0] * 2
    tbl_lin = stage.reshape(Vp, E)

    v = inputs.astype(jnp.int32)
    q = v & (_VBLK - 1)
    vp = (v & ~(_VBLK - 1)) | ((q & (_HALF - 1)) << 1) | (q >> _HBITS)
    idx3 = vp.reshape(NW, CPW, CHUNK_B * S)

    pooled = _make_pool_kernel(B, S, E, Vp, NW, CHUNK_B)(tbl_lin, idx3)

    out = pl.pallas_call(
        _mm_body,
        out_shape=jax.ShapeDtypeStruct((B, O), jnp.float32),
    )(pooled, W.T, b.reshape(1, O))
    return out


# speedup vs baseline: 2.4512x; 1.0005x over previous
import functools

import jax
import jax.numpy as jnp
from jax import lax
from jax.experimental import pallas as pl
from jax.experimental.pallas import tpu as pltpu
from jax.experimental.pallas import tpu_sc as plsc

_HALF = 8192
_HBITS = _HALF.bit_length() - 1
_VBLK = 2 * _HALF


def _tp_body(xt_ref, o_ref):
    x = xt_ref[...]
    E = x.shape[0]
    eye = jnp.eye(E, dtype=jnp.float32)

    def dot_t(m):
        return jax.lax.dot_general(
            m, eye, (((0,), (0,)), ((), ())),
            preferred_element_type=jnp.float32,
        )

    o_ref[...] = jnp.concatenate(
        [dot_t(x[:, :_HALF]), dot_t(x[:, _HALF:])], axis=1
    )


def _stage_table(table_t, V, E):
    nblk = (V + _VBLK - 1) // _VBLK
    return pl.pallas_call(
        _tp_body,
        grid=(nblk,),
        in_specs=[pl.BlockSpec((E, _VBLK), lambda i: (0, i))],
        out_specs=pl.BlockSpec((_HALF, 2 * E), lambda i: (i, 0)),
        out_shape=jax.ShapeDtypeStruct((nblk * _HALF, 2 * E), jnp.float32),
    )(table_t)


def _pool_kernel_body(CPW, CHUNK_B, S, E, table_hbm, idx_hbm, out_hbm,
                      idx_v, rows_v, pool_v, sem):
    NC = 2
    CHUNK_IDX = CHUNK_B * S
    BPW = CPW * CHUNK_B
    wid = lax.axis_index("s") * NC + lax.axis_index("c")

    pltpu.sync_copy(idx_hbm.at[wid], idx_v)

    def _gather(c, par):
        pltpu.async_copy(
            table_hbm.at[idx_v.at[c]],
            rows_v.at[pl.ds(par * CHUNK_IDX, CHUNK_IDX)],
            sem,
        )

    NBUF = 4
    for p in range(NBUF - 1):
        _gather(p, p)

    nvec = E // 16

    def chunk_body(c, carry):
        par = lax.rem(c, NBUF)
        base = par * CHUNK_IDX
        pltpu.make_async_copy(
            table_hbm.at[idx_v.at[c]], rows_v.at[pl.ds(0, CHUNK_IDX)], sem
        ).wait()

        @pl.when(c + NBUF - 1 < CPW)
        def _():
            _gather(c + NBUF - 1, lax.rem(c + NBUF - 1, NBUF))

        for bag in range(CHUNK_B):
            accs = [jnp.zeros((16,), jnp.float32) for _ in range(nvec)]
            for s in range(S):
                r = base + bag * S + s
                for j in range(nvec):
                    accs[j] = accs[j] + rows_v[r, pl.ds(j * 16, 16)]
            row_out = c * CHUNK_B + bag
            for j in range(nvec):
                pool_v[row_out, pl.ds(j * 16, 16)] = accs[j] * (1.0 / S)
        return carry

    lax.fori_loop(0, CPW, chunk_body, 0, unroll=2)

    pltpu.sync_copy(pool_v, out_hbm.at[pl.ds(wid * BPW, BPW)])


def _make_pool_kernel(B, S, E, Vp, NW, CHUNK_B):
    BPW = B // NW
    CPW = BPW // CHUNK_B
    CHUNK_IDX = CHUNK_B * S
    mesh = plsc.VectorSubcoreMesh(core_axis_name="c", subcore_axis_name="s")
    return pl.kernel(
        functools.partial(_pool_kernel_body, CPW, CHUNK_B, S, E),
        out_type=jax.ShapeDtypeStruct((B, E), jnp.float32),
        mesh=mesh,
        scratch_types=[
            pltpu.VMEM((CPW, CHUNK_IDX), jnp.int32),
            pltpu.VMEM((4 * CHUNK_IDX, E), jnp.float32),
            pltpu.VMEM((BPW, E), jnp.float32),
            pltpu.SemaphoreType.DMA,
        ],
        compiler_params=pltpu.CompilerParams(use_tc_tiling_on_sc=False),
    )


def _mm_body(x_ref, w_ref, b_ref, o_ref):
    o_ref[...] = (
        jnp.dot(x_ref[...], w_ref[...], preferred_element_type=jnp.float32)
        + b_ref[...]
    )


def kernel(inputs, table, W, b):
    B, S = inputs.shape
    V, E = table.shape
    O = W.shape[0]
    NW = 32
    CHUNK_B = 2
    BPW = B // NW
    CPW = BPW // CHUNK_B

    stage = _stage_table(table.T, V, E)
    Vp = stage.shape[0] * 2
    tbl_lin = stage.reshape(Vp, E)

    v = inputs.astype(jnp.int32)
    q = v & (_VBLK - 1)
    vp = (v & ~(_VBLK - 1)) | ((q & (_HALF - 1)) << 1) | (q >> _HBITS)
    idx3 = vp.reshape(NW, CPW, CHUNK_B * S)

    pooled = _make_pool_kernel(B, S, E, Vp, NW, CHUNK_B)(tbl_lin, idx3)

    out = pl.pallas_call(
        _mm_body,
        out_shape=jax.ShapeDtypeStruct((B, O), jnp.float32),
    )(pooled, W.T, b.reshape(1, O))
    return out
